# baseline (speedup 1.0000x reference)
import functools

import jax
import jax.numpy as jnp
from jax import lax
from jax.experimental import pallas as pl
from jax.experimental.pallas import tpu as pltpu
from jax.experimental.pallas import tpu_sc as plsc

NUM_EMB = 1002
EMBED = 128
B, L = 4096, 200
N = B * L
NC, NS = 2, 16
NW = NC * NS
PER_W = N // NW
GATHER = 128
NI = PER_W // GATHER
SUB = 2
CHUNK = SUB * GATHER
NJ = PER_W // CHUNK
NBUF = 3


def _emb_body(idx_hbm, w_hbm, out_hbm, idx_v, rows_v, gsem, osem):
    cid = lax.axis_index("c")
    sid = lax.axis_index("s")
    wid = sid * NC + cid
    base = wid * PER_W

    pltpu.sync_copy(idx_hbm.at[wid], idx_v)

    def gather(j, buf):
        for s in range(SUB):
            pltpu.async_copy(
                w_hbm.at[idx_v.at[SUB * j + s]],
                rows_v.at[buf, pl.ds(s * GATHER, GATHER)],
                gsem,
            )

    def put(j, buf):
        pltpu.async_copy(
            rows_v.at[buf], out_hbm.at[pl.ds(base + j * CHUNK, CHUNK)], osem
        )

    def wait_gather(buf):
        for s in range(SUB):
            pltpu.make_async_copy(
                w_hbm.at[idx_v.at[0]],
                rows_v.at[buf, pl.ds(s * GATHER, GATHER)],
                gsem,
            ).wait()

    def wait_put(j, buf):
        pltpu.make_async_copy(
            rows_v.at[buf], out_hbm.at[pl.ds(base + j * CHUNK, CHUNK)], osem
        ).wait()

    gather(0, 0)
    gather(1, 1)
    gather(2, 2)
    wait_gather(0)
    put(0, 0)
    wait_put(0, 0)
    gather(3, 0)
    wait_gather(1)
    put(1, 1)

    @pl.loop(2, NJ - 2, step=NBUF)
    def _steady(j0):
        for b in range(NBUF):
            j = j0 + b
            buf = (j0 + b) % NBUF
            wait_put(j - 1, (buf + 2) % NBUF)
            gather(j + 2, (buf + 2) % NBUF)
            wait_gather(buf)
            put(j, buf)

    wait_put(NJ - 3, (NJ - 3) % NBUF)
    wait_gather((NJ - 2) % NBUF)
    put(NJ - 2, (NJ - 2) % NBUF)
    wait_put(NJ - 2, (NJ - 2) % NBUF)
    wait_gather((NJ - 1) % NBUF)
    put(NJ - 1, (NJ - 1) % NBUF)
    wait_put(NJ - 1, (NJ - 1) % NBUF)


@functools.partial(jax.jit, static_argnames=())
def kernel(indices, W):
    idx = indices.reshape(NW, NI, GATHER)
    mesh = plsc.VectorSubcoreMesh(
        core_axis_name="c", subcore_axis_name="s", num_cores=NC, num_subcores=NS
    )
    run = pl.kernel(
        _emb_body,
        out_type=jax.ShapeDtypeStruct((N, EMBED), jnp.float32),
        mesh=mesh,
        scratch_types=[
            pltpu.VMEM((NI, GATHER), jnp.int32),
            pltpu.VMEM((NBUF, CHUNK, EMBED), jnp.float32),
            pltpu.SemaphoreType.DMA,
            pltpu.SemaphoreType.DMA,
        ],
    )
    out = run(idx, W)
    return out.reshape(B, L, EMBED)


# speedup vs baseline: 2.8694x; 2.8694x over previous
import functools

import jax
import jax.numpy as jnp
from jax import lax
from jax.experimental import pallas as pl
from jax.experimental.pallas import tpu as pltpu
from jax.experimental.pallas import tpu_sc as plsc

NUM_EMB = 1002
EMBED = 128
B, L = 4096, 200
N = B * L
NC, NS = 2, 16
NW = NC * NS
PER_W = N // NW
GATHER = 128
NI = PER_W // GATHER
CHUNK = GATHER
NJ = PER_W // CHUNK
NBUF = 4
AHEAD = 2


def _emb_body(idx_hbm, w_hbm, out_hbm, w_sh, idx_v, rows_v, gsem, osem):
    cid = lax.axis_index("c")
    sid = lax.axis_index("s")
    wid = sid * NC + cid
    base = wid * PER_W

    @pl.when(sid == 0)
    def _():
        pltpu.sync_copy(w_hbm, w_sh)

    pltpu.sync_copy(idx_hbm.at[wid], idx_v)
    plsc.subcore_barrier()

    def gather(j, buf):
        pltpu.async_copy(w_sh.at[idx_v.at[j]], rows_v.at[buf], gsem)

    def put(j, buf):
        pltpu.async_copy(
            rows_v.at[buf], out_hbm.at[pl.ds(base + j * CHUNK, CHUNK)], osem
        )

    def wait_gather(buf):
        pltpu.make_async_copy(w_sh.at[idx_v.at[0]], rows_v.at[buf], gsem).wait()

    def wait_put(j, buf):
        pltpu.make_async_copy(
            rows_v.at[buf], out_hbm.at[pl.ds(base + j * CHUNK, CHUNK)], osem
        ).wait()

    gather(0, 0)
    gather(1, 1)
    gather(2, 2)
    wait_gather(0)
    put(0, 0)
    gather(3, 3)
    wait_gather(1)
    put(1, 1)

    @pl.loop(2, NJ - AHEAD, step=NBUF)
    def _steady(j0):
        for b in range(NBUF):
            j = j0 + b
            buf = (j0 + b) % NBUF
            nbuf = (buf + AHEAD) % NBUF
            wait_put(j - AHEAD, nbuf)
            gather(j + AHEAD, nbuf)
            wait_gather(buf)
            put(j, buf)

    wait_put(NJ - 4, (NJ - 4) % NBUF)
    wait_gather((NJ - 2) % NBUF)
    put(NJ - 2, (NJ - 2) % NBUF)
    wait_put(NJ - 3, (NJ - 3) % NBUF)
    wait_gather((NJ - 1) % NBUF)
    put(NJ - 1, (NJ - 1) % NBUF)
    wait_put(NJ - 2, (NJ - 2) % NBUF)
    wait_put(NJ - 1, (NJ - 1) % NBUF)


@functools.partial(jax.jit, static_argnames=())
def kernel(indices, W):
    idx = indices.reshape(NW, NI, GATHER)
    mesh = plsc.VectorSubcoreMesh(
        core_axis_name="c", subcore_axis_name="s", num_cores=NC, num_subcores=NS
    )
    run = pl.kernel(
        _emb_body,
        out_type=jax.ShapeDtypeStruct((N, EMBED), jnp.float32),
        mesh=mesh,
        scratch_types=[
            pltpu.VMEM_SHARED((NUM_EMB, EMBED), jnp.float32),
            pltpu.VMEM((NI, GATHER), jnp.int32),
            pltpu.VMEM((NBUF, CHUNK, EMBED), jnp.float32),
            pltpu.SemaphoreType.DMA,
            pltpu.SemaphoreType.DMA,
        ],
    )
    out = run(idx, W)
    return out.reshape(B, L, EMBED)
